# baseline (speedup 1.0000x reference)
import functools

import jax
import jax.numpy as jnp
from jax import lax
from jax.experimental import pallas as pl
from jax.experimental.pallas import tpu as pltpu
from jax.experimental.pallas import tpu_sc as plsc

CAPACITY = 262144
OBS_DIM = 128
BATCH = 16384


ROWS = 8192
GRID = CAPACITY // ROWS
NB_BATCH = BATCH // ROWS


def _tc_body(st, ns, stm, nsm, ost, ons):
    i = pl.program_id(0)

    @pl.when(i < NB_BATCH)
    def _():
        ost[...] = st[...]
        ons[...] = ns[...]

    @pl.when(i >= NB_BATCH)
    def _():
        ost[...] = stm[...]
        ons[...] = nsm[...]


def _tc_copy(states, next_states, states_mem, next_states_mem):
    big = pl.BlockSpec((ROWS, OBS_DIM), lambda i: (i, 0))
    big_mem = pl.BlockSpec((ROWS, OBS_DIM), lambda i: (jnp.maximum(i, NB_BATCH), 0))
    big_batch = pl.BlockSpec((ROWS, OBS_DIM), lambda i: (jnp.minimum(i, NB_BATCH - 1), 0))
    return pl.pallas_call(
        _tc_body,
        grid=(GRID,),
        in_specs=[big_batch, big_batch, big_mem, big_mem],
        out_specs=[big, big],
        out_shape=[
            jax.ShapeDtypeStruct((CAPACITY, OBS_DIM), jnp.float32),
            jax.ShapeDtypeStruct((CAPACITY, OBS_DIM), jnp.float32),
        ],
        compiler_params=pltpu.CompilerParams(dimension_semantics=("parallel",)),
    )(states, next_states, states_mem, next_states_mem)


_INFO = plsc.get_sparse_core_info()
_NW = _INFO.num_cores * _INFO.num_subcores
CHUNK = CAPACITY // _NW
NB_W = BATCH // CHUNK


@functools.partial(
    pl.kernel,
    out_type=[
        jax.ShapeDtypeStruct((CAPACITY,), jnp.int32),
        jax.ShapeDtypeStruct((CAPACITY,), jnp.float32),
    ],
    mesh=plsc.VectorSubcoreMesh(core_axis_name="c", subcore_axis_name="s"),
    scratch_types=[
        pltpu.VMEM((CHUNK,), jnp.int32),
        pltpu.VMEM((CHUNK,), jnp.float32),
    ],
)
def _sc_copy(ac, rw, acm, rwm, oac, orw, ac_v, rw_v):
    wid = lax.axis_index("s") * _INFO.num_cores + lax.axis_index("c")
    base = wid * CHUNK

    @pl.when(wid < NB_W)
    def _():
        pltpu.sync_copy(ac.at[pl.ds(base, CHUNK)], ac_v)
        pltpu.sync_copy(rw.at[pl.ds(base, CHUNK)], rw_v)

    @pl.when(wid >= NB_W)
    def _():
        pltpu.sync_copy(acm.at[pl.ds(base, CHUNK)], ac_v)
        pltpu.sync_copy(rwm.at[pl.ds(base, CHUNK)], rw_v)

    pltpu.sync_copy(ac_v, oac.at[pl.ds(base, CHUNK)])
    pltpu.sync_copy(rw_v, orw.at[pl.ds(base, CHUNK)])


def kernel(states, actions, next_states, rewards, states_mem, next_states_mem, actions_mem, rewards_mem):
    out_ac, out_rw = _sc_copy(actions, rewards, actions_mem, rewards_mem)
    out_st, out_ns = _tc_copy(states, next_states, states_mem, next_states_mem)
    return (out_st, out_ac, out_ns, out_rw)


# speedup vs baseline: 1.0006x; 1.0006x over previous
import functools

import jax
import jax.numpy as jnp
from jax import lax
from jax.experimental import pallas as pl
from jax.experimental.pallas import tpu as pltpu
from jax.experimental.pallas import tpu_sc as plsc

CAPACITY = 262144
OBS_DIM = 128
BATCH = 16384


ROWS = 8192
GRID = CAPACITY // ROWS
NB_BATCH = BATCH // ROWS


def _tc_body(st, ns, stm, nsm, ost, ons):
    i = pl.program_id(0)

    @pl.when(i < NB_BATCH)
    def _():
        ost[...] = st[...]
        ons[...] = ns[...]

    @pl.when(i >= NB_BATCH)
    def _():
        ost[...] = stm[...]
        ons[...] = nsm[...]


def _tc_copy(states, next_states, states_mem, next_states_mem):
    big = pl.BlockSpec((ROWS, OBS_DIM), lambda i: (i, 0))
    big_mem = pl.BlockSpec((ROWS, OBS_DIM), lambda i: (jnp.maximum(i, NB_BATCH), 0))
    big_batch = pl.BlockSpec((ROWS, OBS_DIM), lambda i: (jnp.minimum(i, NB_BATCH - 1), 0))
    return pl.pallas_call(
        _tc_body,
        grid=(GRID,),
        in_specs=[big_batch, big_batch, big_mem, big_mem],
        out_specs=[big, big],
        out_shape=[
            jax.ShapeDtypeStruct((CAPACITY, OBS_DIM), jnp.float32),
            jax.ShapeDtypeStruct((CAPACITY, OBS_DIM), jnp.float32),
        ],
        compiler_params=pltpu.CompilerParams(dimension_semantics=("parallel",)),
    )(states, next_states, states_mem, next_states_mem)


_INFO = plsc.get_sparse_core_info()
_NW = _INFO.num_cores * _INFO.num_subcores
CHUNK = CAPACITY // _NW
NB_W = BATCH // CHUNK


@functools.partial(
    pl.kernel,
    out_type=[
        jax.ShapeDtypeStruct((CAPACITY,), jnp.int32),
        jax.ShapeDtypeStruct((CAPACITY,), jnp.float32),
    ],
    mesh=plsc.VectorSubcoreMesh(core_axis_name="c", subcore_axis_name="s"),
    scratch_types=[
        pltpu.VMEM((CHUNK,), jnp.int32),
        pltpu.VMEM((CHUNK,), jnp.float32),
    ],
)
def _sc_copy(ac, rw, acm, rwm, oac, orw, ac_v, rw_v):
    wid = lax.axis_index("s") * _INFO.num_cores + lax.axis_index("c")
    base = wid * CHUNK

    @pl.when(wid < NB_W)
    def _():
        pltpu.sync_copy(ac.at[pl.ds(base, CHUNK)], ac_v)
        pltpu.sync_copy(rw.at[pl.ds(base, CHUNK)], rw_v)

    @pl.when(wid >= NB_W)
    def _():
        pltpu.sync_copy(acm.at[pl.ds(base, CHUNK)], ac_v)
        pltpu.sync_copy(rwm.at[pl.ds(base, CHUNK)], rw_v)

    pltpu.sync_copy(ac_v, oac.at[pl.ds(base, CHUNK)])
    pltpu.sync_copy(rw_v, orw.at[pl.ds(base, CHUNK)])


def kernel(states, actions, next_states, rewards, states_mem, next_states_mem, actions_mem, rewards_mem):
    out_st, out_ns = _tc_copy(states, next_states, states_mem, next_states_mem)
    out_ac, out_rw = _sc_copy(actions, rewards, actions_mem, rewards_mem)
    return (out_st, out_ac, out_ns, out_rw)


# speedup vs baseline: 1.0909x; 1.0903x over previous
import jax
import jax.numpy as jnp
from jax.experimental import pallas as pl
from jax.experimental.pallas import tpu as pltpu

CAPACITY = 262144
OBS_DIM = 128
BATCH = 16384

ROWS = 8192
GRID = CAPACITY // ROWS
NB_BATCH = BATCH // ROWS
SROWS = ROWS // 128


def _body(st, ac, ns, rw, stm, acm, nsm, rwm, ost, oac, ons, orw):
    i = pl.program_id(0)

    @pl.when(i < NB_BATCH)
    def _():
        ost[...] = st[...]
        oac[...] = ac[...]
        ons[...] = ns[...]
        orw[...] = rw[...]

    @pl.when(i >= NB_BATCH)
    def _():
        ost[...] = stm[...]
        oac[...] = acm[...]
        ons[...] = nsm[...]
        orw[...] = rwm[...]


def kernel(states, actions, next_states, rewards, states_mem, next_states_mem, actions_mem, rewards_mem):
    ac2 = actions.reshape(BATCH // 128, 128)
    rw2 = rewards.reshape(BATCH // 128, 128)
    acm2 = actions_mem.reshape(CAPACITY // 128, 128)
    rwm2 = rewards_mem.reshape(CAPACITY // 128, 128)

    big = pl.BlockSpec((ROWS, OBS_DIM), lambda i: (i, 0))
    small = pl.BlockSpec((SROWS, 128), lambda i: (i, 0))
    big_mem = pl.BlockSpec((ROWS, OBS_DIM), lambda i: (jnp.maximum(i, NB_BATCH), 0))
    small_mem = pl.BlockSpec((SROWS, 128), lambda i: (jnp.maximum(i, NB_BATCH), 0))
    big_batch = pl.BlockSpec((ROWS, OBS_DIM), lambda i: (jnp.minimum(i, NB_BATCH - 1), 0))
    small_batch = pl.BlockSpec((SROWS, 128), lambda i: (jnp.minimum(i, NB_BATCH - 1), 0))

    out_st, out_ac2, out_ns, out_rw2 = pl.pallas_call(
        _body,
        grid=(GRID,),
        in_specs=[big_batch, small_batch, big_batch, small_batch,
                  big_mem, small_mem, big_mem, small_mem],
        out_specs=[big, small, big, small],
        out_shape=[
            jax.ShapeDtypeStruct((CAPACITY, OBS_DIM), jnp.float32),
            jax.ShapeDtypeStruct((CAPACITY // 128, 128), jnp.int32),
            jax.ShapeDtypeStruct((CAPACITY, OBS_DIM), jnp.float32),
            jax.ShapeDtypeStruct((CAPACITY // 128, 128), jnp.float32),
        ],
        compiler_params=pltpu.CompilerParams(dimension_semantics=("parallel",)),
    )(states, ac2, next_states, rw2, states_mem, acm2, next_states_mem, rwm2)

    return (out_st, out_ac2.reshape(CAPACITY), out_ns, out_rw2.reshape(CAPACITY))
